# baseline (speedup 1.0000x reference)
import dataclasses
import functools

import jax
import jax.numpy as jnp
from jax import lax
from jax.experimental import pallas as pl
from jax.experimental.pallas import tpu as pltpu
from jax.experimental.pallas import tpu_sc as plsc

N = 10000
E = 160000
D = 256
H = 512

NC = 2
NS = 16
LANES = 16
NCHUNK = 4
CW = 128
BE = 128
GBE = 64
EW = 10240
NB = EW // GBE
EPAD = NS * EW

NPASS = 2
PASS_SPLIT = 5120
GARBAGE = 5120
ACC_ROWS = 5128
ZRPS = 320


def _proj_body(x_ref, ws_ref, wd_ref, wl_ref, atts_ref, attd_ref, bl_ref,
               xs4_ref, as_ref, ad_ref, xlin_ref):
    x = x_ref[...]
    xs = jnp.dot(x, ws_ref[...], preferred_element_type=jnp.float32)
    xd = jnp.dot(x, wd_ref[...], preferred_element_type=jnp.float32)
    xlin_ref[...] = (
        jnp.dot(x, wl_ref[...], preferred_element_type=jnp.float32)
        + bl_ref[...][None, :])
    as_ref[...] = jnp.dot(xs, atts_ref[...],
                          preferred_element_type=jnp.float32)[None, None, :]
    ad_ref[...] = jnp.dot(xd, attd_ref[...],
                          preferred_element_type=jnp.float32)[None, None, :]
    for c in range(NCHUNK):
        xs4_ref[c] = xs[:, c * CW:(c + 1) * CW]


def _proj(x, Wsrc, Wdst, att_src, att_dst, Wlin, blin):
    din = x.shape[1]
    nb = 1000
    grid = (N // nb,)
    return pl.pallas_call(
        _proj_body,
        grid=grid,
        in_specs=[
            pl.BlockSpec((nb, din), lambda i: (i, 0)),
            pl.BlockSpec((din, H), lambda i: (0, 0)),
            pl.BlockSpec((din, H), lambda i: (0, 0)),
            pl.BlockSpec((din, H), lambda i: (0, 0)),
            pl.BlockSpec((H,), lambda i: (0,)),
            pl.BlockSpec((H,), lambda i: (0,)),
            pl.BlockSpec((H,), lambda i: (0,)),
        ],
        out_specs=[
            pl.BlockSpec((NCHUNK, nb, CW), lambda i: (0, i, 0)),
            pl.BlockSpec((1, 1, nb), lambda i: (i, 0, 0)),
            pl.BlockSpec((1, 1, nb), lambda i: (i, 0, 0)),
            pl.BlockSpec((nb, H), lambda i: (i, 0)),
        ],
        out_shape=[
            jax.ShapeDtypeStruct((NCHUNK, N, CW), jnp.float32),
            jax.ShapeDtypeStruct((N // nb, 1, nb), jnp.float32),
            jax.ShapeDtypeStruct((N // nb, 1, nb), jnp.float32),
            jax.ShapeDtypeStruct((N, H), jnp.float32),
        ],
    )(x, Wsrc, Wdst, Wlin, att_src, att_dst, blin)


def _sc_edge_body(xs_a, xs_b, xs_c, xs_d, as_hbm, ad_hbm, bound_hbm,
                  src_hbm, dst_hbm,
                  s_hbm, den_hbm,
                  asv, adv, boundv, srcv, dstv, idxb, g0, g1,
                  acc_sh, sem0, sem1):
    xs_chunks = [xs_a, xs_b, xs_c, xs_d]
    cid = lax.axis_index("c")
    sid = lax.axis_index("s")

    pltpu.sync_copy(as_hbm, asv)
    pltpu.sync_copy(ad_hbm, adv)
    pltpu.sync_copy(bound_hbm, boundv)
    pltpu.sync_copy(src_hbm.at[sid], srcv)
    pltpu.sync_copy(dst_hbm.at[sid], dstv)

    zero16 = jnp.zeros((LANES,), jnp.float32)
    bound = boundv[...]
    iota = lax.iota(jnp.int32, LANES)
    ebase = sid * EW

    def compute_ex16(b, k):
        s16 = srcv[b, pl.ds(k, LANES)]
        d16 = dstv[b, pl.ds(k, LANES)]
        av = plsc.load_gather(asv, [s16]) + plsc.load_gather(adv, [d16])
        av = jnp.where(av >= 0.0, av, 0.2 * av)
        ex = jnp.exp(av - bound)
        gid = ebase + b * GBE + k + iota
        return jnp.where(gid < E, ex, 0.0)

    def zero_g0():
        @pl.loop(0, GBE)
        def _(r):
            for g in range(CW // LANES):
                g0[r, pl.ds(g * LANES, LANES)] = zero16

    def zero_acc():
        zero_g0()
        base = sid * ZRPS
        for o in range(0, ZRPS, GBE):
            pltpu.sync_copy(g0, acc_sh.at[pl.ds(base + o, GBE)])

        @pl.when(sid == NS - 1)
        def _():
            pltpu.sync_copy(g0.at[pl.ds(0, ACC_ROWS - 16 * ZRPS)],
                            acc_sh.at[pl.ds(16 * ZRPS, ACC_ROWS - 16 * ZRPS)])

        plsc.subcore_barrier()

    def write_acc(out2d, p, lo):
        plsc.subcore_barrier()

        @pl.when(p == 0)
        def _():
            pltpu.sync_copy(acc_sh.at[pl.ds(sid * 320, 320)],
                            out2d.at[pl.ds(sid * 320, 320)])

        @pl.when(jnp.logical_and(p == 1, sid < NS - 1))
        def _():
            pltpu.sync_copy(acc_sh.at[pl.ds(sid * 304, 304)],
                            out2d.at[pl.ds(lo + sid * 304, 304)])

        @pl.when(jnp.logical_and(p == 1, sid == NS - 1))
        def _():
            pltpu.sync_copy(acc_sh.at[pl.ds(4560, 320)],
                            out2d.at[pl.ds(lo + 4560, 320)])

        plsc.subcore_barrier()

    def adjust_idx(row, b, lo, hi):
        @pl.loop(0, GBE, step=LANES)
        def _(k):
            d16 = dstv[b, pl.ds(k, LANES)]
            ok = jnp.logical_and(d16 >= lo, d16 < hi)
            idxb[row, pl.ds(k, LANES)] = jnp.where(ok, d16 - lo, GARBAGE)

    def scale_rows(buf, b):
        @pl.loop(0, GBE, step=LANES)
        def _(r0):
            ex16 = compute_ex16(b, r0)
            for j in range(LANES):
                s = ex16[j]
                for g in range(CW // LANES):
                    sl = pl.ds(g * LANES, LANES)
                    buf[r0 + j, sl] = buf[r0 + j, sl] * s

    def fill_rows(buf, b):
        @pl.loop(0, GBE, step=LANES)
        def _(r0):
            ex16 = compute_ex16(b, r0)
            for j in range(LANES):
                s = ex16[j]
                for g in range(CW // LANES):
                    buf[r0 + j, pl.ds(g * LANES, LANES)] = s + zero16

    for c in range(NCHUNK):
        @pl.when(cid == c // 2)
        def _(c=c):
            data = xs_chunks[c]

            @pl.loop(0, NPASS)
            def _(p):
                lo = p * PASS_SPLIT
                hi = jnp.minimum(lo + PASS_SPLIT, N)
                zero_acc()

                @pl.loop(0, NB, step=2)
                def _(b):
                    cp0 = pltpu.async_copy(data.at[srcv.at[b]], g0, sem0)
                    cp1 = pltpu.async_copy(data.at[srcv.at[b + 1]], g1, sem1)
                    adjust_idx(0, b, lo, hi)
                    adjust_idx(1, b + 1, lo, hi)
                    cp0.wait()
                    scale_rows(g0, b)
                    pltpu.sync_copy(g0, acc_sh.at[idxb.at[0]], add=True)
                    cp1.wait()
                    scale_rows(g1, b + 1)
                    pltpu.sync_copy(g1, acc_sh.at[idxb.at[1]], add=True)

                write_acc(s_hbm.at[c], p, lo)

    @pl.loop(0, NPASS)
    def _(p):
        lo = p * PASS_SPLIT
        hi = jnp.minimum(lo + PASS_SPLIT, N)
        zero_acc()

        @pl.loop(0, NB // 2)
        def _(i):
            b = cid * (NB // 2) + i
            fill_rows(g0, b)
            adjust_idx(0, b, lo, hi)
            pltpu.sync_copy(g0, acc_sh.at[idxb.at[0]], add=True)

        @pl.when(cid == 0)
        def _():
            write_acc(den_hbm.at[0], p, lo)

        @pl.when(cid == 1)
        def _():
            write_acc(den_hbm.at[1], p, lo)


def _sc_edge(xs4, a_s, a_d, bound16, src3, dst3):
    cp = pltpu.CompilerParams()
    if "needs_layout_passes" in pltpu.CompilerParams.__dataclass_fields__:
        cp = dataclasses.replace(cp, needs_layout_passes=False)
    kern = pl.kernel(
        _sc_edge_body,
        out_type=[
            jax.ShapeDtypeStruct((NCHUNK, N, CW), jnp.float32),
            jax.ShapeDtypeStruct((NC, N, CW), jnp.float32),
        ],
        mesh=plsc.VectorSubcoreMesh(core_axis_name="c", subcore_axis_name="s"),
        scratch_types=[
            pltpu.VMEM((N,), jnp.float32),
            pltpu.VMEM((N,), jnp.float32),
            pltpu.VMEM((LANES,), jnp.float32),
            pltpu.VMEM((NB, GBE), jnp.int32),
            pltpu.VMEM((NB, GBE), jnp.int32),
            pltpu.VMEM((2, GBE), jnp.int32),
            pltpu.VMEM((GBE, CW), jnp.float32),
            pltpu.VMEM((GBE, CW), jnp.float32),
            pltpu.VMEM_SHARED((ACC_ROWS, CW), jnp.float32),
            pltpu.SemaphoreType.DMA,
            pltpu.SemaphoreType.DMA,
        ],
        compiler_params=cp,
    )
    return kern(xs4[0], xs4[1], xs4[2], xs4[3], a_s, a_d, bound16,
                src3, dst3)


def _epi_body(s_ref, den_ref, xlin_ref, bias_ref, o_ref, *, relu):
    d = den_ref[0, :, 0:1] + den_ref[1, :, 0:1] + 1e-16
    for c in range(NCHUNK):
        sl = slice(c * CW, (c + 1) * CW)
        v = s_ref[c] / d + xlin_ref[:, sl] + bias_ref[...][None, sl]
        if relu:
            v = jnp.maximum(v, 0.0)
        o_ref[:, sl] = v


def _epilogue(s, den, xlin, bias, relu):
    nb = 1000
    return pl.pallas_call(
        functools.partial(_epi_body, relu=relu),
        grid=(N // nb,),
        in_specs=[
            pl.BlockSpec((NCHUNK, nb, CW), lambda i: (0, i, 0)),
            pl.BlockSpec((NC, nb, CW), lambda i: (0, i, 0)),
            pl.BlockSpec((nb, H), lambda i: (i, 0)),
            pl.BlockSpec((H,), lambda i: (0,)),
        ],
        out_specs=pl.BlockSpec((nb, H), lambda i: (i, 0)),
        out_shape=jax.ShapeDtypeStruct((N, H), jnp.float32),
    )(s, den, xlin, bias)


def _layer(x, src3, dst3, Wsrc, Wdst, att_src, att_dst, bias, Wlin, blin,
           relu):
    xs4, a_s, a_d, xlin = _proj(x, Wsrc, Wdst, att_src, att_dst, Wlin, blin)
    b = jnp.max(a_s) + jnp.max(a_d)
    b = jnp.where(b >= 0.0, b, 0.2 * b)
    bound16 = jnp.full((LANES,), b, jnp.float32)
    s, den = _sc_edge(xs4, a_s.reshape(N), a_d.reshape(N), bound16,
                      src3, dst3)
    return _epilogue(s, den, xlin, bias, relu)


def kernel(x, edge_index, Wsrc1, Wdst1, att_src1, att_dst1, bias1, Wlin1,
           blin1, Wsrc2, Wdst2, att_src2, att_dst2, bias2, Wlin2, blin2):
    src = edge_index[0]
    dst = edge_index[1]
    pad = jnp.zeros((EPAD - E,), jnp.int32)
    src3 = jnp.concatenate([src, pad]).reshape(NS, NB, GBE)
    dst3 = jnp.concatenate([dst, pad]).reshape(NS, NB, GBE)
    h = _layer(x, src3, dst3, Wsrc1, Wdst1, att_src1, att_dst1, bias1,
               Wlin1, blin1, relu=True)
    return _layer(h, src3, dst3, Wsrc2, Wdst2, att_src2, att_dst2, bias2,
                  Wlin2, blin2, relu=False)


# speedup vs baseline: 1.0586x; 1.0586x over previous
import dataclasses
import functools

import jax
import jax.numpy as jnp
from jax import lax
from jax.experimental import pallas as pl
from jax.experimental.pallas import tpu as pltpu
from jax.experimental.pallas import tpu_sc as plsc

N = 10000
E = 160000
D = 256
H = 512

NC = 2
NS = 16
LANES = 16
NCHUNK = 4
CW = 128
BE = 128
EW = 10240
NB = EW // BE
EPAD = NS * EW

NPASS = 2
PASS_SPLIT = 5120
GARBAGE = 5120
ACC_ROWS = 5128
ZRPS = 320


def _proj_body(x_ref, ws_ref, wd_ref, wl_ref, atts_ref, attd_ref, bl_ref,
               xs4_ref, as_ref, ad_ref, xlin_ref):
    x = x_ref[...]
    xs = jnp.dot(x, ws_ref[...], preferred_element_type=jnp.float32)
    xd = jnp.dot(x, wd_ref[...], preferred_element_type=jnp.float32)
    xlin_ref[...] = (
        jnp.dot(x, wl_ref[...], preferred_element_type=jnp.float32)
        + bl_ref[...][None, :])
    as_ref[...] = jnp.dot(xs, atts_ref[...],
                          preferred_element_type=jnp.float32)[None, None, :]
    ad_ref[...] = jnp.dot(xd, attd_ref[...],
                          preferred_element_type=jnp.float32)[None, None, :]
    for c in range(NCHUNK):
        xs4_ref[c] = xs[:, c * CW:(c + 1) * CW]


def _proj(x, Wsrc, Wdst, att_src, att_dst, Wlin, blin):
    din = x.shape[1]
    nb = 1000
    grid = (N // nb,)
    return pl.pallas_call(
        _proj_body,
        grid=grid,
        in_specs=[
            pl.BlockSpec((nb, din), lambda i: (i, 0)),
            pl.BlockSpec((din, H), lambda i: (0, 0)),
            pl.BlockSpec((din, H), lambda i: (0, 0)),
            pl.BlockSpec((din, H), lambda i: (0, 0)),
            pl.BlockSpec((H,), lambda i: (0,)),
            pl.BlockSpec((H,), lambda i: (0,)),
            pl.BlockSpec((H,), lambda i: (0,)),
        ],
        out_specs=[
            pl.BlockSpec((NCHUNK, nb, CW), lambda i: (0, i, 0)),
            pl.BlockSpec((1, 1, nb), lambda i: (i, 0, 0)),
            pl.BlockSpec((1, 1, nb), lambda i: (i, 0, 0)),
            pl.BlockSpec((nb, H), lambda i: (i, 0)),
        ],
        out_shape=[
            jax.ShapeDtypeStruct((NCHUNK, N, CW), jnp.float32),
            jax.ShapeDtypeStruct((N // nb, 1, nb), jnp.float32),
            jax.ShapeDtypeStruct((N // nb, 1, nb), jnp.float32),
            jax.ShapeDtypeStruct((N, H), jnp.float32),
        ],
    )(x, Wsrc, Wdst, Wlin, att_src, att_dst, blin)


def _sc_edge_body(xs_a, xs_b, xs_c, xs_d, as_hbm, ad_hbm, bound_hbm,
                  src_hbm, dst_hbm,
                  s_hbm, den_hbm,
                  asv, adv, boundv, srcv, dstv, idxb, g0,
                  acc_sh, sem0):
    xs_chunks = [xs_a, xs_b, xs_c, xs_d]
    cid = lax.axis_index("c")
    sid = lax.axis_index("s")

    pltpu.sync_copy(as_hbm, asv)
    pltpu.sync_copy(ad_hbm, adv)
    pltpu.sync_copy(bound_hbm, boundv)
    pltpu.sync_copy(src_hbm.at[sid], srcv)
    pltpu.sync_copy(dst_hbm.at[sid], dstv)

    zero16 = jnp.zeros((LANES,), jnp.float32)
    bound = boundv[...]
    iota = lax.iota(jnp.int32, LANES)
    ebase = sid * EW

    def compute_ex16(b, k):
        s16 = srcv[b, pl.ds(k, LANES)]
        d16 = dstv[b, pl.ds(k, LANES)]
        av = plsc.load_gather(asv, [s16]) + plsc.load_gather(adv, [d16])
        av = jnp.where(av >= 0.0, av, 0.2 * av)
        ex = jnp.exp(av - bound)
        gid = ebase + b * BE + k + iota
        return jnp.where(gid < E, ex, 0.0)

    def zero_g0():
        @pl.loop(0, BE)
        def _(r):
            for g in range(CW // LANES):
                g0[r, pl.ds(g * LANES, LANES)] = zero16

    def zero_acc():
        zero_g0()
        base = sid * ZRPS
        pltpu.sync_copy(g0, acc_sh.at[pl.ds(base, BE)])
        pltpu.sync_copy(g0, acc_sh.at[pl.ds(base + BE, BE)])

        @pl.when(sid < NS - 1)
        def _():
            pltpu.sync_copy(g0.at[pl.ds(0, ZRPS - 2 * BE)],
                            acc_sh.at[pl.ds(base + 2 * BE, ZRPS - 2 * BE)])

        @pl.when(sid == NS - 1)
        def _():
            pltpu.sync_copy(g0.at[pl.ds(0, ACC_ROWS - 15 * ZRPS - 2 * BE)],
                            acc_sh.at[pl.ds(base + 2 * BE,
                                            ACC_ROWS - 15 * ZRPS - 2 * BE)])

        plsc.subcore_barrier()

    def write_acc(out2d, p, lo):
        plsc.subcore_barrier()

        @pl.when(p == 0)
        def _():
            pltpu.sync_copy(acc_sh.at[pl.ds(sid * 320, 320)],
                            out2d.at[pl.ds(sid * 320, 320)])

        @pl.when(jnp.logical_and(p == 1, sid < NS - 1))
        def _():
            pltpu.sync_copy(acc_sh.at[pl.ds(sid * 304, 304)],
                            out2d.at[pl.ds(lo + sid * 304, 304)])

        @pl.when(jnp.logical_and(p == 1, sid == NS - 1))
        def _():
            pltpu.sync_copy(acc_sh.at[pl.ds(4560, 320)],
                            out2d.at[pl.ds(lo + 4560, 320)])

        plsc.subcore_barrier()

    def adjust_idx(row, b, lo, hi):
        @pl.loop(0, BE, step=LANES)
        def _(k):
            d16 = dstv[b, pl.ds(k, LANES)]
            ok = jnp.logical_and(d16 >= lo, d16 < hi)
            idxb[row, pl.ds(k, LANES)] = jnp.where(ok, d16 - lo, GARBAGE)

    def scale_rows(buf, b):
        @pl.loop(0, BE, step=LANES)
        def _(r0):
            ex16 = compute_ex16(b, r0)
            for j in range(LANES):
                s = ex16[j]
                for g in range(CW // LANES):
                    sl = pl.ds(g * LANES, LANES)
                    buf[r0 + j, sl] = buf[r0 + j, sl] * s

    def fill_rows(buf, b):
        @pl.loop(0, BE, step=LANES)
        def _(r0):
            ex16 = compute_ex16(b, r0)
            for j in range(LANES):
                s = ex16[j]
                for g in range(CW // LANES):
                    buf[r0 + j, pl.ds(g * LANES, LANES)] = s + zero16

    for c in range(NCHUNK):
        @pl.when(cid == c // 2)
        def _(c=c):
            data = xs_chunks[c]

            @pl.loop(0, NPASS)
            def _(p):
                lo = p * PASS_SPLIT
                hi = jnp.minimum(lo + PASS_SPLIT, N)
                zero_acc()

                @pl.loop(0, NB)
                def _(b):
                    cp0 = pltpu.async_copy(data.at[srcv.at[b]], g0, sem0)
                    adjust_idx(0, b, lo, hi)
                    cp0.wait()
                    scale_rows(g0, b)
                    pltpu.sync_copy(g0, acc_sh.at[idxb.at[0]], add=True)

                write_acc(s_hbm.at[c], p, lo)

    @pl.loop(0, NPASS)
    def _(p):
        lo = p * PASS_SPLIT
        hi = jnp.minimum(lo + PASS_SPLIT, N)
        zero_acc()

        @pl.loop(0, NB // 2)
        def _(i):
            b = cid * (NB // 2) + i
            fill_rows(g0, b)
            adjust_idx(0, b, lo, hi)
            pltpu.sync_copy(g0, acc_sh.at[idxb.at[0]], add=True)

        @pl.when(cid == 0)
        def _():
            write_acc(den_hbm.at[0], p, lo)

        @pl.when(cid == 1)
        def _():
            write_acc(den_hbm.at[1], p, lo)


def _sc_edge(xs4, a_s, a_d, bound16, src3, dst3):
    cp = pltpu.CompilerParams()
    if "needs_layout_passes" in pltpu.CompilerParams.__dataclass_fields__:
        cp = dataclasses.replace(cp, needs_layout_passes=False)
    kern = pl.kernel(
        _sc_edge_body,
        out_type=[
            jax.ShapeDtypeStruct((NCHUNK, N, CW), jnp.float32),
            jax.ShapeDtypeStruct((NC, N, CW), jnp.float32),
        ],
        mesh=plsc.VectorSubcoreMesh(core_axis_name="c", subcore_axis_name="s"),
        scratch_types=[
            pltpu.VMEM((N,), jnp.float32),
            pltpu.VMEM((N,), jnp.float32),
            pltpu.VMEM((LANES,), jnp.float32),
            pltpu.VMEM((NB, BE), jnp.int32),
            pltpu.VMEM((NB, BE), jnp.int32),
            pltpu.VMEM((2, BE), jnp.int32),
            pltpu.VMEM((BE, CW), jnp.float32),
            pltpu.VMEM_SHARED((ACC_ROWS, CW), jnp.float32),
            pltpu.SemaphoreType.DMA,
        ],
        compiler_params=cp,
    )
    return kern(xs4[0], xs4[1], xs4[2], xs4[3], a_s, a_d, bound16,
                src3, dst3)


def _epi_body(s_ref, den_ref, xlin_ref, bias_ref, o_ref, *, relu):
    d = den_ref[0, :, 0:1] + den_ref[1, :, 0:1] + 1e-16
    for c in range(NCHUNK):
        sl = slice(c * CW, (c + 1) * CW)
        v = s_ref[c] / d + xlin_ref[:, sl] + bias_ref[...][None, sl]
        if relu:
            v = jnp.maximum(v, 0.0)
        o_ref[:, sl] = v


def _epilogue(s, den, xlin, bias, relu):
    nb = 1000
    return pl.pallas_call(
        functools.partial(_epi_body, relu=relu),
        grid=(N // nb,),
        in_specs=[
            pl.BlockSpec((NCHUNK, nb, CW), lambda i: (0, i, 0)),
            pl.BlockSpec((NC, nb, CW), lambda i: (0, i, 0)),
            pl.BlockSpec((nb, H), lambda i: (i, 0)),
            pl.BlockSpec((H,), lambda i: (0,)),
        ],
        out_specs=pl.BlockSpec((nb, H), lambda i: (i, 0)),
        out_shape=jax.ShapeDtypeStruct((N, H), jnp.float32),
    )(s, den, xlin, bias)


def _layer(x, src3, dst3, Wsrc, Wdst, att_src, att_dst, bias, Wlin, blin,
           relu):
    xs4, a_s, a_d, xlin = _proj(x, Wsrc, Wdst, att_src, att_dst, Wlin, blin)
    b = jnp.max(a_s) + jnp.max(a_d)
    b = jnp.where(b >= 0.0, b, 0.2 * b)
    bound16 = jnp.full((LANES,), b, jnp.float32)
    s, den = _sc_edge(xs4, a_s.reshape(N), a_d.reshape(N), bound16,
                      src3, dst3)
    return _epilogue(s, den, xlin, bias, relu)


def kernel(x, edge_index, Wsrc1, Wdst1, att_src1, att_dst1, bias1, Wlin1,
           blin1, Wsrc2, Wdst2, att_src2, att_dst2, bias2, Wlin2, blin2):
    src = edge_index[0]
    dst = edge_index[1]
    pad = jnp.zeros((EPAD - E,), jnp.int32)
    src3 = jnp.concatenate([src, pad]).reshape(NS, NB, BE)
    dst3 = jnp.concatenate([dst, pad]).reshape(NS, NB, BE)
    h = _layer(x, src3, dst3, Wsrc1, Wdst1, att_src1, att_dst1, bias1,
               Wlin1, blin1, relu=True)
    return _layer(h, src3, dst3, Wsrc2, Wdst2, att_src2, att_dst2, bias2,
                  Wlin2, blin2, relu=False)
